# Initial kernel scaffold; baseline (speedup 1.0000x reference)
import jax
import jax.numpy as jnp
from jax.experimental import pallas as pl


def kernel(digits, table):
    raise NotImplementedError("write your pallas kernel here")


# speedup vs baseline: 153.9305x; 153.9305x over previous
import functools

import jax
import jax.numpy as jnp
from jax import lax
from jax.experimental import pallas as pl
from jax.experimental.pallas import tpu as pltpu
from jax.experimental.pallas import tpu_sc as plsc

NC = 2
NS = 16
L = 16
NW = NC * NS
UNROLL = 4


def _make_kernel(n, vocab, embed):
    assert embed == L
    assert n % (NW * L * UNROLL) == 0
    n_w = n // NW
    n_vec = n_w // L
    inv_n = 1.0 / n

    mesh = plsc.VectorSubcoreMesh(
        core_axis_name="c", subcore_axis_name="s",
        num_cores=NC, num_subcores=NS)

    @functools.partial(
        pl.kernel,
        out_type=jax.ShapeDtypeStruct((NW, L), jnp.float32),
        mesh=mesh,
        compiler_params=pltpu.CompilerParams(needs_layout_passes=False),
        scratch_types=[
            pltpu.VMEM((n_w,), jnp.int32),
            pltpu.VMEM((vocab, L), jnp.float32),
            pltpu.VMEM((UNROLL * L * L,), jnp.int32),
            pltpu.VMEM((L,), jnp.float32),
        ],
    )
    def k(digits_hbm, table_hbm, out_hbm, chunk_v, table_v, hist_v, out_v):
        wid = lax.axis_index("s") * NC + lax.axis_index("c")
        base = wid * n_w
        pltpu.sync_copy(digits_hbm.at[pl.ds(base, n_w)], chunk_v)
        pltpu.sync_copy(table_hbm, table_v)

        zero16 = jnp.zeros((L,), jnp.int32)
        for u in range(UNROLL):
            for d in range(vocab):
                hist_v[pl.ds((u * L + d) * L, L)] = zero16

        lane = lax.iota(jnp.int32, 16)
        ones = jnp.ones((L,), jnp.int32)
        lane_off = [lane + u * L * L for u in range(UNROLL)]

        @plsc.parallel_loop(0, n_vec, step=UNROLL, unroll=2)
        def _(i):
            for u in range(UNROLL):
                v = chunk_v[pl.ds((i + u) * L, L)]
                idx = (v << 4) + lane_off[u]
                plsc.addupdate_scatter(hist_v, [idx], ones)

        accf = jnp.zeros((L,), jnp.float32)
        for d in range(vocab):
            row = hist_v[pl.ds(d * L, L)]
            for u in range(1, UNROLL):
                row = row + hist_v[pl.ds((u * L + d) * L, L)]
            total = jnp.sum(row).astype(jnp.float32)
            accf = accf + total * table_v[d, :]
        out_v[:] = accf * inv_n
        pltpu.sync_copy(out_v, out_hbm.at[wid])

    return k


def kernel(digits, table):
    n, = digits.shape
    vocab, embed = table.shape
    parts = _make_kernel(n, vocab, embed)(digits.astype(jnp.int32), table)
    return parts.sum(axis=0)
